# baseline (speedup 1.0000x reference)
import functools

import jax
import jax.numpy as jnp
from jax import lax
from jax.experimental import pallas as pl
from jax.experimental.pallas import tpu as pltpu
from jax.experimental.pallas import tpu_sc as plsc

_INFO = plsc.get_sparse_core_info()
_NC = _INFO.num_cores
_NS = _INFO.num_subcores
_NW = _NC * _NS

_IW = 128
_G = 8


def _gather_impl(table, idx2d, n_rows, embed):
    n_per_w = n_rows // _NW
    blk = _G * _IW
    n_outer = n_per_w // blk

    mesh = plsc.VectorSubcoreMesh(core_axis_name="c", subcore_axis_name="s")

    @functools.partial(
        pl.kernel,
        mesh=mesh,
        compiler_params=pltpu.CompilerParams(use_tc_tiling_on_sc=False),
        out_type=jax.ShapeDtypeStruct((n_rows, embed), jnp.float32),
        scratch_types=[
            pltpu.VMEM((_G, _IW), jnp.int32),
            pltpu.VMEM((blk, embed), jnp.float32),
            pltpu.SemaphoreType.DMA,
        ],
    )
    def k(table_hbm, idx_hbm, out_hbm, idx_v, rows_v, sem):
        wid = lax.axis_index("s") * _NC + lax.axis_index("c")
        row_base = wid * n_per_w

        def outer(i, carry):
            row_off = pl.multiple_of(row_base + i * blk, blk)
            pltpu.sync_copy(
                idx_hbm.at[pl.ds(pl.multiple_of(row_off // _IW, _G), _G)],
                idx_v,
            )
            waits = []
            for g in range(_G):
                waits.append(
                    pltpu.async_copy(
                        table_hbm.at[idx_v.at[g]],
                        rows_v.at[pl.ds(g * _IW, _IW)],
                        sem,
                    )
                )
            for w in waits:
                w.wait()
            pltpu.sync_copy(rows_v, out_hbm.at[pl.ds(row_off, blk)])
            return carry

        lax.fori_loop(0, n_outer, outer, 0)

    return k(table, idx2d)


def kernel(words, feats, table):
    batch, seq = words.shape
    vocab, embed = table.shape
    n_rows = batch * seq
    idx2d = words.reshape(n_rows // _IW, _IW)
    out = _gather_impl(table, idx2d, n_rows, embed)
    return out.reshape(batch, seq, embed)


# speedup vs baseline: 1.0187x; 1.0187x over previous
import functools

import jax
import jax.numpy as jnp
from jax import lax
from jax.experimental import pallas as pl
from jax.experimental.pallas import tpu as pltpu
from jax.experimental.pallas import tpu_sc as plsc

_INFO = plsc.get_sparse_core_info()
_NC = _INFO.num_cores
_NS = _INFO.num_subcores
_NW = _NC * _NS

_IW = 128
_G = 5


def _gather_impl(table, idx2d, n_rows, embed):
    n_per_w = n_rows // _NW
    iw_per_w = n_per_w // _IW
    blk = _G * _IW
    n_blocks = n_per_w // blk
    assert n_blocks % 2 == 0 and n_blocks * blk == n_per_w

    mesh = plsc.VectorSubcoreMesh(core_axis_name="c", subcore_axis_name="s")

    @functools.partial(
        pl.kernel,
        mesh=mesh,
        compiler_params=pltpu.CompilerParams(use_tc_tiling_on_sc=False),
        out_type=jax.ShapeDtypeStruct((n_rows, embed), jnp.float32),
        scratch_types=[
            pltpu.VMEM((iw_per_w, _IW), jnp.int32),
            pltpu.VMEM((blk, embed), jnp.float32),
            pltpu.VMEM((blk, embed), jnp.float32),
            pltpu.SemaphoreType.DMA,
            pltpu.SemaphoreType.DMA,
            pltpu.SemaphoreType.DMA,
            pltpu.SemaphoreType.DMA,
        ],
    )
    def k(table_hbm, idx_hbm, out_hbm, idx_all, rows0, rows1,
          sem_g0, sem_g1, sem_o0, sem_o1):
        wid = lax.axis_index("s") * _NC + lax.axis_index("c")
        row_base = wid * n_per_w
        rows = (rows0, rows1)
        sem_g = (sem_g0, sem_g1)
        sem_o = (sem_o0, sem_o1)

        pltpu.sync_copy(
            idx_hbm.at[pl.ds(pl.multiple_of(wid * iw_per_w, 8), iw_per_w)],
            idx_all,
        )

        def outer(jj, carry):
            for g in range(2):
                j = jj * 2 + g
                row_off = pl.multiple_of(row_base + j * blk, 8)

                @pl.when(jj > 0)
                def _drain_store():
                    pltpu.make_async_copy(
                        out_hbm.at[pl.ds(0, blk)], rows[g], sem_o[g]
                    ).wait()

                waits = []
                for t in range(_G):
                    waits.append(
                        pltpu.async_copy(
                            table_hbm.at[idx_all.at[j * _G + t]],
                            rows[g].at[pl.ds(t * _IW, _IW)],
                            sem_g[g],
                        )
                    )
                for w in waits:
                    w.wait()

                pltpu.async_copy(
                    rows[g], out_hbm.at[pl.ds(row_off, blk)], sem_o[g]
                )
            return carry

        lax.fori_loop(0, n_blocks // 2, outer, 0)

        for g in range(2):
            pltpu.make_async_copy(
                out_hbm.at[pl.ds(0, blk)], rows[g], sem_o[g]
            ).wait()

    return k(table, idx2d)


def kernel(words, feats, table):
    batch, seq = words.shape
    vocab, embed = table.shape
    n_rows = batch * seq
    idx2d = words.reshape(n_rows // _IW, _IW)
    out = _gather_impl(table, idx2d, n_rows, embed)
    return out.reshape(batch, seq, embed)
